# Initial kernel scaffold; baseline (speedup 1.0000x reference)
import jax
import jax.numpy as jnp
from jax.experimental import pallas as pl


def kernel(x, w, b):
    raise NotImplementedError("write your pallas kernel here")


# speedup vs baseline: 1.1348x; 1.1348x over previous
import functools

import jax
import jax.numpy as jnp
from jax.experimental import pallas as pl
from jax.experimental.pallas import tpu as pltpu


def _conv1x1_body(x_ref, w_ref, b_ref, o_ref):
    acc = jax.lax.dot_general(
        w_ref[...], x_ref[...],
        dimension_numbers=(((1,), (0,)), ((), ())),
        preferred_element_type=jnp.float32,
    )
    o_ref[...] = (acc + b_ref[...]).astype(o_ref.dtype)


@functools.partial(jax.jit, static_argnames=("tile_hw",))
def _outconv(x, w, b, *, tile_hw=32768):
    N, C_in, H, W = x.shape
    C_out = w.shape[0]
    HW = H * W

    x3 = x.reshape(N, C_in, HW)
    w2 = w.reshape(C_out, C_in).astype(jnp.float32)
    b2 = b.reshape(C_out, 1).astype(jnp.float32)

    tm = HW if HW <= tile_hw else tile_hw
    num_m = pl.cdiv(HW, tm)
    grid = (num_m, N)

    x_bytes = 2 * C_in * tm * x.dtype.itemsize
    o_bytes = 2 * C_out * tm * x.dtype.itemsize
    vmem_limit = int(min(x_bytes + o_bytes + (8 << 20), 56 << 20))

    out3 = pl.pallas_call(
        _conv1x1_body,
        out_shape=jax.ShapeDtypeStruct((N, C_out, HW), x.dtype),
        grid=grid,
        in_specs=[
            pl.BlockSpec((None, C_in, tm), lambda m, n: (n, 0, m)),
            pl.BlockSpec((C_out, C_in), lambda m, n: (0, 0)),
            pl.BlockSpec((C_out, 1), lambda m, n: (0, 0)),
        ],
        out_specs=pl.BlockSpec((None, C_out, tm), lambda m, n: (n, 0, m)),
        compiler_params=pltpu.CompilerParams(
            dimension_semantics=("parallel", "parallel"),
            vmem_limit_bytes=vmem_limit,
        ),
    )(x3, w2, b2)

    return out3.reshape(N, C_out, H, W)


def kernel(x, w, b):
    return _outconv(x, w, b)


# speedup vs baseline: 3.6593x; 3.2246x over previous
import functools

import jax
import jax.numpy as jnp
from jax.experimental import pallas as pl
from jax.experimental.pallas import tpu as pltpu


def _make_body(c_in, c_out):
    def body(x_ref, w_ref, b_ref, o_ref):
        x = x_ref[...]
        for o in range(c_out):
            acc = x[0] * w_ref[o, 0] + b_ref[o]
            for c in range(1, c_in):
                acc = acc + x[c] * w_ref[o, c]
            o_ref[o] = acc.astype(o_ref.dtype)
    return body


@functools.partial(jax.jit, static_argnames=("tile_h",))
def _outconv4d(x, w, b, *, tile_h=128):
    N, C_in, H, W = x.shape
    C_out = w.shape[0]

    w2 = w.reshape(C_out, C_in).astype(jnp.float32)
    b1 = b.astype(jnp.float32)

    th = H if H <= tile_h else tile_h
    num_h = pl.cdiv(H, th)
    grid = (N, num_h)

    x_bytes = 2 * C_in * th * W * x.dtype.itemsize
    o_bytes = 2 * C_out * th * W * x.dtype.itemsize
    vmem_limit = int(min(x_bytes + o_bytes + (8 << 20), 56 << 20))

    out = pl.pallas_call(
        _make_body(C_in, C_out),
        out_shape=jax.ShapeDtypeStruct((N, C_out, H, W), x.dtype),
        grid=grid,
        in_specs=[
            pl.BlockSpec((None, C_in, th, W), lambda n, h: (n, 0, h, 0)),
            pl.BlockSpec(memory_space=pltpu.MemorySpace.SMEM),
            pl.BlockSpec(memory_space=pltpu.MemorySpace.SMEM),
        ],
        out_specs=pl.BlockSpec((None, C_out, th, W), lambda n, h: (n, 0, h, 0)),
        compiler_params=pltpu.CompilerParams(
            dimension_semantics=("parallel", "parallel"),
            vmem_limit_bytes=vmem_limit,
        ),
    )(x, w2, b1)

    return out


def kernel(x, w, b):
    return _outconv4d(x, w, b)
